# Initial kernel scaffold; baseline (speedup 1.0000x reference)
import jax
import jax.numpy as jnp
from jax.experimental import pallas as pl


def kernel(inputs, lengths, attention_vector, weight, bias):
    raise NotImplementedError("write your pallas kernel here")


# speedup vs baseline: 1.1561x; 1.1561x over previous
import jax
import jax.numpy as jnp
from jax.experimental import pallas as pl
from jax.experimental.pallas import tpu as pltpu

_SUBLANE = 8


def _attn_pool_head_kernel(x_ref, len_ref, v_ref, w_ref, b_ref, out_ref):
    x = x_ref[...]
    TB, S, H = x.shape

    xr = x.reshape(TB * S, H)
    logits = jax.lax.dot_general(
        xr, v_ref[...],
        dimension_numbers=(((1,), (1,)), ((), ())),
        preferred_element_type=jnp.float32,
    ).reshape(TB, S, 1)

    m = jnp.max(logits, axis=1, keepdims=True)
    un = jnp.exp(logits - m)

    t = jax.lax.broadcasted_iota(jnp.int32, (TB, S, 1), 1)
    lens = len_ref[...].reshape(TB, 1, 1)
    w_s = jnp.where(t < lens, un, 0.0)

    denom = jnp.sum(w_s, axis=1)
    rep_un = jnp.sum(x * w_s, axis=1)
    rep = rep_un * pl.reciprocal(denom, approx=False)

    out_ref[...] = jax.lax.dot_general(
        rep, w_ref[...],
        dimension_numbers=(((1,), (1,)), ((), ())),
        preferred_element_type=jnp.float32,
    ) + b_ref[...]


def kernel(inputs, lengths, attention_vector, weight, bias):
    B, S, H = inputs.shape
    NI = weight.shape[0]

    TB = 32
    while TB > _SUBLANE and B % TB != 0:
        TB //= 2
    n_tiles = pl.cdiv(B, TB)
    B_pad = n_tiles * TB

    x = inputs.astype(jnp.float32)
    lens = lengths.astype(jnp.int32)
    if B_pad != B:
        x = jnp.pad(x, ((0, B_pad - B), (0, 0), (0, 0)))
        lens = jnp.pad(lens, (0, B_pad - B), constant_values=1)
    lens_2d = lens.reshape(B_pad, 1)
    v_2d = attention_vector.reshape(1, H).astype(jnp.float32)
    w = weight.astype(jnp.float32)
    b_2d = bias.reshape(1, NI).astype(jnp.float32)

    x_tile_bytes = TB * S * H * 4
    cost = pl.CostEstimate(
        flops=int(4 * B_pad * S * H + 2 * B_pad * H * NI),
        transcendentals=int(B_pad * S),
        bytes_accessed=int(B_pad * S * H * 4 + (NI * H + NI + H) * 4
                           + B_pad * NI * 4),
    )

    out = pl.pallas_call(
        _attn_pool_head_kernel,
        out_shape=jax.ShapeDtypeStruct((B_pad, NI), jnp.float32),
        grid=(n_tiles,),
        in_specs=[
            pl.BlockSpec((TB, S, H), lambda i: (i, 0, 0)),
            pl.BlockSpec((TB, 1), lambda i: (i, 0)),
            pl.BlockSpec((1, H), lambda i: (0, 0)),
            pl.BlockSpec((NI, H), lambda i: (0, 0)),
            pl.BlockSpec((1, NI), lambda i: (0, 0)),
        ],
        out_specs=pl.BlockSpec((TB, NI), lambda i: (i, 0)),
        compiler_params=pltpu.CompilerParams(
            dimension_semantics=("parallel",),
            vmem_limit_bytes=int(min(96 * 1024 * 1024, 8 * x_tile_bytes)),
        ),
        cost_estimate=cost,
    )(x, lens_2d, v_2d, w, b_2d)

    return out[:B] if B_pad != B else out


# speedup vs baseline: 1.1666x; 1.0091x over previous
import jax
import jax.numpy as jnp
from jax.experimental import pallas as pl
from jax.experimental.pallas import tpu as pltpu

_SUBLANE = 8


_N_SPLIT = 2


def _attn_pool_head_kernel(*refs):
    x_refs = refs[:_N_SPLIT]
    len_ref, v_ref, w_ref, b_ref, out_ref = refs[_N_SPLIT:]
    for k, x_ref in enumerate(x_refs):
        x = x_ref[...]
        TBh, S, H = x.shape

        xr = x.reshape(TBh * S, H)
        logits = jax.lax.dot_general(
            xr, v_ref[...],
            dimension_numbers=(((1,), (1,)), ((), ())),
            preferred_element_type=jnp.float32,
        ).reshape(TBh, S, 1)

        m = jnp.max(logits, axis=1, keepdims=True)
        un = jnp.exp(logits - m)

        t = jax.lax.broadcasted_iota(jnp.int32, (TBh, S, 1), 1)
        lens = len_ref[k * TBh:(k + 1) * TBh, :].reshape(TBh, 1, 1)
        w_s = jnp.where(t < lens, un, 0.0)

        denom = jnp.sum(w_s, axis=1)
        rep_un = jnp.sum(x * w_s, axis=1)
        rep = rep_un * pl.reciprocal(denom, approx=False)

        out_ref[k * TBh:(k + 1) * TBh, :] = jax.lax.dot_general(
            rep, w_ref[...],
            dimension_numbers=(((1,), (1,)), ((), ())),
            preferred_element_type=jnp.float32,
        ) + b_ref[...]


def kernel(inputs, lengths, attention_vector, weight, bias):
    B, S, H = inputs.shape
    NI = weight.shape[0]

    TB = _SUBLANE * _N_SPLIT
    for cand_tbh in (16, 8):
        t = cand_tbh * _N_SPLIT
        if B % t == 0 or B <= t:
            TB = t
            break
    n_tiles = pl.cdiv(B, TB)
    B_pad = n_tiles * TB

    x = inputs.astype(jnp.float32)
    lens = lengths.astype(jnp.int32)
    if B_pad != B:
        x = jnp.pad(x, ((0, B_pad - B), (0, 0), (0, 0)))
        lens = jnp.pad(lens, (0, B_pad - B), constant_values=1)
    lens_2d = lens.reshape(B_pad, 1)
    v_2d = attention_vector.reshape(1, H).astype(jnp.float32)
    w = weight.astype(jnp.float32)
    b_2d = bias.reshape(1, NI).astype(jnp.float32)

    x_tile_bytes = TB * S * H * 4
    cost = pl.CostEstimate(
        flops=int(4 * B_pad * S * H + 2 * B_pad * H * NI),
        transcendentals=int(B_pad * S),
        bytes_accessed=int(B_pad * S * H * 4 + (NI * H + NI + H) * 4
                           + B_pad * NI * 4),
    )

    TBh = TB // _N_SPLIT
    x_specs = [
        pl.BlockSpec((TBh, S, H), lambda i, k=k: (_N_SPLIT * i + k, 0, 0))
        for k in range(_N_SPLIT)
    ]
    out = pl.pallas_call(
        _attn_pool_head_kernel,
        out_shape=jax.ShapeDtypeStruct((B_pad, NI), jnp.float32),
        grid=(n_tiles,),
        in_specs=x_specs + [
            pl.BlockSpec((TB, 1), lambda i: (i, 0)),
            pl.BlockSpec((1, H), lambda i: (0, 0)),
            pl.BlockSpec((NI, H), lambda i: (0, 0)),
            pl.BlockSpec((1, NI), lambda i: (0, 0)),
        ],
        out_specs=pl.BlockSpec((TB, NI), lambda i: (i, 0)),
        compiler_params=pltpu.CompilerParams(
            dimension_semantics=("parallel",),
            vmem_limit_bytes=int(min(96 * 1024 * 1024, 8 * x_tile_bytes)),
        ),
        cost_estimate=cost,
    )(*([x] * _N_SPLIT), lens_2d, v_2d, w, b_2d)

    return out[:B] if B_pad != B else out


# speedup vs baseline: 1.2734x; 1.0915x over previous
import jax
import jax.numpy as jnp
from jax.experimental import pallas as pl
from jax.experimental.pallas import tpu as pltpu

_TB = 32
_NBUF = 4
_CORES = 2


def _chunk_compute(x, lens, v_ref, w_ref, b_ref):
    TB, S, H = x.shape

    xr = x.reshape(TB * S, H)
    logits = jax.lax.dot_general(
        xr, v_ref[...],
        dimension_numbers=(((1,), (1,)), ((), ())),
        preferred_element_type=jnp.float32,
    ).reshape(TB, S, 1)

    m = jnp.max(logits, axis=1, keepdims=True)
    un = jnp.exp(logits - m)

    t = jax.lax.broadcasted_iota(jnp.int32, (TB, S, 1), 1)
    w_s = jnp.where(t < lens.reshape(TB, 1, 1), un, 0.0)

    denom = jnp.sum(w_s, axis=1)
    rep_un = jnp.sum(x * w_s, axis=1)
    rep = rep_un * pl.reciprocal(denom, approx=False)

    return jax.lax.dot_general(
        rep, w_ref[...],
        dimension_numbers=(((1,), (1,)), ((), ())),
        preferred_element_type=jnp.float32,
    ) + b_ref[...]


def _attn_pool_head_kernel(x_hbm, len_ref, v_ref, w_ref, b_ref, out_ref,
                           buf, sem):
    rows = out_ref.shape[0]
    n_chunks = rows // _TB
    base = pl.program_id(0) * rows

    def _issue(c, slot):
        pltpu.make_async_copy(
            x_hbm.at[pl.ds(base + c * _TB, _TB)],
            buf.at[slot],
            sem.at[slot],
        ).start()

    for c in range(min(_NBUF, n_chunks)):
        _issue(c, c)

    def _step(c, carry):
        slot = jax.lax.rem(c, _NBUF)
        pltpu.make_async_copy(
            x_hbm.at[pl.ds(base + c * _TB, _TB)],
            buf.at[slot],
            sem.at[slot],
        ).wait()
        x = buf[slot]
        lens = len_ref[pl.ds(c * _TB, _TB), :]
        out_ref[pl.ds(c * _TB, _TB), :] = _chunk_compute(
            x, lens, v_ref, w_ref, b_ref)
        nxt = c + _NBUF

        @pl.when(nxt < n_chunks)
        def _():
            _issue(nxt, slot)

        return carry

    jax.lax.fori_loop(0, n_chunks, _step, 0)


def kernel(inputs, lengths, attention_vector, weight, bias):
    B, S, H = inputs.shape
    NI = weight.shape[0]

    chunk_rows = _CORES * _TB
    B_pad = ((B + chunk_rows - 1) // chunk_rows) * chunk_rows
    rows = B_pad // _CORES

    x = inputs.astype(jnp.float32)
    lens = lengths.astype(jnp.int32)
    if B_pad != B:
        x = jnp.pad(x, ((0, B_pad - B), (0, 0), (0, 0)))
        lens = jnp.pad(lens, (0, B_pad - B), constant_values=1)
    lens_2d = lens.reshape(B_pad, 1)
    v_2d = attention_vector.reshape(1, H).astype(jnp.float32)
    w = weight.astype(jnp.float32)
    b_2d = bias.reshape(1, NI).astype(jnp.float32)

    chunk_bytes = _TB * S * H * 4
    cost = pl.CostEstimate(
        flops=int(4 * B_pad * S * H + 2 * B_pad * H * NI),
        transcendentals=int(B_pad * S),
        bytes_accessed=int(B_pad * S * H * 4 + (NI * H + NI + H) * 4
                           + B_pad * NI * 4),
    )

    out = pl.pallas_call(
        _attn_pool_head_kernel,
        out_shape=jax.ShapeDtypeStruct((B_pad, NI), jnp.float32),
        grid=(_CORES,),
        in_specs=[
            pl.BlockSpec(memory_space=pl.ANY),
            pl.BlockSpec((rows, 1), lambda i: (i, 0)),
            pl.BlockSpec((1, H), lambda i: (0, 0)),
            pl.BlockSpec((NI, H), lambda i: (0, 0)),
            pl.BlockSpec((1, NI), lambda i: (0, 0)),
        ],
        out_specs=pl.BlockSpec((rows, NI), lambda i: (i, 0)),
        scratch_shapes=[
            pltpu.VMEM((_NBUF, _TB, S, H), jnp.float32),
            pltpu.SemaphoreType.DMA((_NBUF,)),
        ],
        compiler_params=pltpu.CompilerParams(
            dimension_semantics=("parallel",),
            vmem_limit_bytes=int(min(100 * 1024 * 1024,
                                     (_NBUF + 4) * chunk_bytes)),
        ),
        cost_estimate=cost,
    )(x, lens_2d, v_2d, w, b_2d)

    return out[:B] if B_pad != B else out
